# baseline (speedup 1.0000x reference)
import jax
import jax.numpy as jnp
from jax import lax
from jax.experimental import pallas as pl
from jax.experimental.pallas import tpu as pltpu
from jax.experimental.pallas import tpu_sc as plsc

N_CORES = 2
N_SUBCORES = 16
NW = N_CORES * N_SUBCORES

B = 16384
FPT_DIM = 256
DIM = 64
B_PER_W = B // NW
CHUNK = 128
N_CHUNKS = B_PER_W // CHUNK


def _sc_drug_body(didx_hbm, demb_hbm, drugs_out, didx_v, dbuf0, dbuf1, sem0):
  wid = lax.axis_index("s") * N_CORES + lax.axis_index("c")
  base = wid * B_PER_W

  for k in range(N_CHUNKS):
    pltpu.sync_copy(didx_hbm.at[pl.ds(base + k * CHUNK, CHUNK)], didx_v.at[k])

  bufs = (dbuf0, dbuf1)
  cps = [None] * N_CHUNKS
  cps[0] = pltpu.async_copy(demb_hbm.at[didx_v.at[0]], bufs[0], sem0)
  for k in range(N_CHUNKS):
    if k + 1 < N_CHUNKS:
      cps[k + 1] = pltpu.async_copy(
          demb_hbm.at[didx_v.at[k + 1]], bufs[(k + 1) % 2], sem0)
    cps[k].wait()
    pltpu.sync_copy(bufs[k % 2],
                    drugs_out.at[pl.ds(base + k * CHUNK, CHUNK)])


def _sc_adr_body(didx_hbm, aidx_hbm, aemb_hbm, bd_hbm, ba_hbm,
                 adrs_out, bd_out, ba_out,
                 didx_v, aidx_v, abuf, bdbuf, babuf, sem1, sem2):
  wid = lax.axis_index("s") * N_CORES + lax.axis_index("c")
  base = wid * B_PER_W

  for k in range(N_CHUNKS):
    pltpu.sync_copy(didx_hbm.at[pl.ds(base + k * CHUNK, CHUNK)], didx_v.at[k])
    pltpu.sync_copy(aidx_hbm.at[pl.ds(base + k * CHUNK, CHUNK)], aidx_v.at[k])

  cps = []
  for k in range(N_CHUNKS):
    cps.append(pltpu.async_copy(
        aemb_hbm.at[aidx_v.at[k]], abuf.at[pl.ds(k * CHUNK, CHUNK)], sem1))
  for k in range(N_CHUNKS):
    cps.append(pltpu.async_copy(
        bd_hbm.at[didx_v.at[k]], bdbuf.at[pl.ds(k * CHUNK, CHUNK)], sem2))
    cps.append(pltpu.async_copy(
        ba_hbm.at[aidx_v.at[k]], babuf.at[pl.ds(k * CHUNK, CHUNK)], sem2))
  for cp in cps:
    cp.wait()
  pltpu.sync_copy(abuf, adrs_out.at[pl.ds(base, B_PER_W)])
  pltpu.sync_copy(bdbuf, bd_out.at[pl.ds(base, B_PER_W)])
  pltpu.sync_copy(babuf, ba_out.at[pl.ds(base, B_PER_W)])


def _sc_gather(drug_idx, adr_idx, drug_embeddings, adr_embeddings, bias_d,
               bias_a):
  mesh = plsc.VectorSubcoreMesh(core_axis_name="c", subcore_axis_name="s")

  drug_fn = pl.kernel(
      _sc_drug_body,
      out_type=jax.ShapeDtypeStruct((B, FPT_DIM), jnp.float32),
      mesh=mesh,
      scratch_types=[
          pltpu.VMEM((8, CHUNK), jnp.int32),
          pltpu.VMEM((CHUNK, FPT_DIM), jnp.float32),
          pltpu.VMEM((CHUNK, FPT_DIM), jnp.float32),
          pltpu.SemaphoreType.DMA,
      ],
      compiler_params=pltpu.CompilerParams(use_tc_tiling_on_sc=True))
  drugs_g = drug_fn(drug_idx, drug_embeddings)

  adr_fn = pl.kernel(
      _sc_adr_body,
      out_type=(
          jax.ShapeDtypeStruct((B, DIM), jnp.float32),
          jax.ShapeDtypeStruct((B,), jnp.float32),
          jax.ShapeDtypeStruct((B,), jnp.float32),
      ),
      mesh=mesh,
      scratch_types=[
          pltpu.VMEM((N_CHUNKS, CHUNK), jnp.int32),
          pltpu.VMEM((N_CHUNKS, CHUNK), jnp.int32),
          pltpu.VMEM((B_PER_W, DIM), jnp.float32),
          pltpu.VMEM((B_PER_W,), jnp.float32),
          pltpu.VMEM((B_PER_W,), jnp.float32),
          pltpu.SemaphoreType.DMA,
          pltpu.SemaphoreType.DMA,
      ],
      compiler_params=pltpu.CompilerParams(use_tc_tiling_on_sc=False))
  adrs_g, bd_g, ba_g = adr_fn(drug_idx, adr_idx, adr_embeddings,
                              bias_d[:, 0], bias_a[:, 0])
  return drugs_g, adrs_g, bd_g, ba_g


def _tc_score_body(drugs_ref, adrs_ref, bd_ref, ba_ref, lw_ref, lb_ref,
                   out_ref):
  proj = lax.dot_general(drugs_ref[...], lw_ref[...],
                         (((1,), (1,)), ((), ())),
                         preferred_element_type=jnp.float32)
  proj = proj + lb_ref[...]
  s = jnp.sum(proj * adrs_ref[...], axis=1)
  out_ref[...] = s + bd_ref[...] + ba_ref[...]


def _tc_score(drugs_g, adrs_g, bd_g, ba_g, L_w, L_b):
  blk = 2048
  grid = (B // blk,)
  return pl.pallas_call(
      _tc_score_body,
      grid=grid,
      in_specs=[
          pl.BlockSpec((blk, FPT_DIM), lambda i: (i, 0)),
          pl.BlockSpec((blk, DIM), lambda i: (i, 0)),
          pl.BlockSpec((blk,), lambda i: (i,)),
          pl.BlockSpec((blk,), lambda i: (i,)),
          pl.BlockSpec((DIM, FPT_DIM), lambda i: (0, 0)),
          pl.BlockSpec((1, DIM), lambda i: (0, 0)),
      ],
      out_specs=pl.BlockSpec((blk,), lambda i: (i,)),
      out_shape=jax.ShapeDtypeStruct((B,), jnp.float32),
  )(drugs_g, adrs_g, bd_g, ba_g, L_w, L_b.reshape(1, DIM))


def kernel(drug_idx, adr_idx, drug_embeddings, adr_embeddings, bias_d, bias_a,
           L_w, L_b):
  drug_idx = drug_idx.astype(jnp.int32)
  adr_idx = adr_idx.astype(jnp.int32)
  drugs_g, adrs_g, bd_g, ba_g = _sc_gather(
      drug_idx, adr_idx, drug_embeddings, adr_embeddings, bias_d, bias_a)
  return _tc_score(drugs_g, adrs_g, bd_g, ba_g, L_w, L_b)


# speedup vs baseline: 1.0350x; 1.0350x over previous
import jax
import jax.numpy as jnp
from jax import lax
from jax.experimental import pallas as pl
from jax.experimental.pallas import tpu as pltpu
from jax.experimental.pallas import tpu_sc as plsc

N_CORES = 2
N_SUBCORES = 16
NW = N_CORES * N_SUBCORES

B = 16384
FPT_DIM = 256
DIM = 64
B_PER_W = B // NW
CHUNK = 128
N_CHUNKS = B_PER_W // CHUNK
LANES = 16


def _sc_drug_body(didx_hbm, demb_hbm, drugs_out, didx_v, dbuf0, dbuf1, sem0):
  wid = lax.axis_index("s") * N_CORES + lax.axis_index("c")
  base = wid * B_PER_W

  for k in range(N_CHUNKS):
    pltpu.sync_copy(didx_hbm.at[pl.ds(base + k * CHUNK, CHUNK)], didx_v.at[k])

  bufs = (dbuf0, dbuf1)
  cps = [None] * N_CHUNKS
  cps[0] = pltpu.async_copy(demb_hbm.at[didx_v.at[0]], bufs[0], sem0)
  for k in range(N_CHUNKS):
    if k + 1 < N_CHUNKS:
      cps[k + 1] = pltpu.async_copy(
          demb_hbm.at[didx_v.at[k + 1]], bufs[(k + 1) % 2], sem0)
    cps[k].wait()
    pltpu.sync_copy(bufs[k % 2],
                    drugs_out.at[pl.ds(base + k * CHUNK, CHUNK)])


def _sc_adr_body(didx_hbm, aidx_hbm, aemb2_hbm, bd_hbm, ba_hbm,
                 pairs_out, bd_out, ba_out,
                 didx_v, aidx_v, pidx_v, pbuf, bdbuf, babuf, sem1, sem2):
  wid = lax.axis_index("s") * N_CORES + lax.axis_index("c")
  base = wid * B_PER_W

  for k in range(N_CHUNKS):
    pltpu.sync_copy(didx_hbm.at[pl.ds(base + k * CHUNK, CHUNK)], didx_v.at[k])
    pltpu.sync_copy(aidx_hbm.at[pl.ds(base + k * CHUNK, CHUNK)], aidx_v.at[k])

  for k in range(N_CHUNKS):
    for j in range(CHUNK // LANES):
      sl = pl.ds(j * LANES, LANES)
      pidx_v[k, sl] = lax.shift_right_logical(aidx_v[k, sl], 1)

  cps = []
  for k in range(N_CHUNKS):
    cps.append(pltpu.async_copy(
        aemb2_hbm.at[pidx_v.at[k]], pbuf.at[pl.ds(k * CHUNK, CHUNK)], sem1))
  for k in range(N_CHUNKS):
    cps.append(pltpu.async_copy(
        bd_hbm.at[didx_v.at[k]], bdbuf.at[pl.ds(k * CHUNK, CHUNK)], sem2))
    cps.append(pltpu.async_copy(
        ba_hbm.at[aidx_v.at[k]], babuf.at[pl.ds(k * CHUNK, CHUNK)], sem2))
  for cp in cps:
    cp.wait()
  pltpu.sync_copy(pbuf, pairs_out.at[pl.ds(base, B_PER_W)])
  pltpu.sync_copy(bdbuf, bd_out.at[pl.ds(base, B_PER_W)])
  pltpu.sync_copy(babuf, ba_out.at[pl.ds(base, B_PER_W)])


def _sc_gather(drug_idx, adr_idx, drug_embeddings, adr_embeddings, bias_d,
               bias_a):
  mesh = plsc.VectorSubcoreMesh(core_axis_name="c", subcore_axis_name="s")

  drug_fn = pl.kernel(
      _sc_drug_body,
      out_type=jax.ShapeDtypeStruct((B, FPT_DIM), jnp.float32),
      mesh=mesh,
      scratch_types=[
          pltpu.VMEM((8, CHUNK), jnp.int32),
          pltpu.VMEM((CHUNK, FPT_DIM), jnp.float32),
          pltpu.VMEM((CHUNK, FPT_DIM), jnp.float32),
          pltpu.SemaphoreType.DMA,
      ],
      compiler_params=pltpu.CompilerParams(use_tc_tiling_on_sc=True))
  drugs_g = drug_fn(drug_idx, drug_embeddings)

  adr_fn = pl.kernel(
      _sc_adr_body,
      out_type=(
          jax.ShapeDtypeStruct((B, 2 * DIM), jnp.float32),
          jax.ShapeDtypeStruct((B,), jnp.float32),
          jax.ShapeDtypeStruct((B,), jnp.float32),
      ),
      mesh=mesh,
      scratch_types=[
          pltpu.VMEM((8, CHUNK), jnp.int32),
          pltpu.VMEM((8, CHUNK), jnp.int32),
          pltpu.VMEM((8, CHUNK), jnp.int32),
          pltpu.VMEM((B_PER_W, 2 * DIM), jnp.float32),
          pltpu.VMEM((B_PER_W,), jnp.float32),
          pltpu.VMEM((B_PER_W,), jnp.float32),
          pltpu.SemaphoreType.DMA,
          pltpu.SemaphoreType.DMA,
      ],
      compiler_params=pltpu.CompilerParams(use_tc_tiling_on_sc=True))
  n_adr = adr_embeddings.shape[0]
  pairs_g, bd_g, ba_g = adr_fn(
      drug_idx, adr_idx, adr_embeddings.reshape(n_adr // 2, 2 * DIM),
      bias_d[:, 0], bias_a[:, 0])
  return drugs_g, pairs_g, bd_g, ba_g


def _tc_score_body(drugs_ref, pairs_ref, aidx_ref, bd_ref, ba_ref, lw_ref,
                   lb_ref, out_ref):
  proj = lax.dot_general(drugs_ref[...], lw_ref[...],
                         (((1,), (1,)), ((), ())),
                         preferred_element_type=jnp.float32)
  proj = proj + lb_ref[...]
  pair = pairs_ref[...]
  s_lo = jnp.sum(proj * pair[:, :DIM], axis=1)
  s_hi = jnp.sum(proj * pair[:, DIM:], axis=1)
  s = jnp.where((aidx_ref[...] & 1) == 1, s_hi, s_lo)
  out_ref[...] = s + bd_ref[...] + ba_ref[...]


def _tc_score(drugs_g, pairs_g, adr_idx, bd_g, ba_g, L_w, L_b):
  blk = 2048
  grid = (B // blk,)
  return pl.pallas_call(
      _tc_score_body,
      grid=grid,
      in_specs=[
          pl.BlockSpec((blk, FPT_DIM), lambda i: (i, 0)),
          pl.BlockSpec((blk, 2 * DIM), lambda i: (i, 0)),
          pl.BlockSpec((blk,), lambda i: (i,)),
          pl.BlockSpec((blk,), lambda i: (i,)),
          pl.BlockSpec((blk,), lambda i: (i,)),
          pl.BlockSpec((DIM, FPT_DIM), lambda i: (0, 0)),
          pl.BlockSpec((1, DIM), lambda i: (0, 0)),
      ],
      out_specs=pl.BlockSpec((blk,), lambda i: (i,)),
      out_shape=jax.ShapeDtypeStruct((B,), jnp.float32),
  )(drugs_g, pairs_g, adr_idx, bd_g, ba_g, L_w, L_b.reshape(1, DIM))


def kernel(drug_idx, adr_idx, drug_embeddings, adr_embeddings, bias_d, bias_a,
           L_w, L_b):
  drug_idx = drug_idx.astype(jnp.int32)
  adr_idx = adr_idx.astype(jnp.int32)
  drugs_g, pairs_g, bd_g, ba_g = _sc_gather(
      drug_idx, adr_idx, drug_embeddings, adr_embeddings, bias_d, bias_a)
  return _tc_score(drugs_g, pairs_g, adr_idx, bd_g, ba_g, L_w, L_b)


# speedup vs baseline: 1.0980x; 1.0608x over previous
import jax
import jax.numpy as jnp
from jax import lax
from jax.experimental import pallas as pl
from jax.experimental.pallas import tpu as pltpu
from jax.experimental.pallas import tpu_sc as plsc

N_CORES = 2
N_SUBCORES = 16
NW = N_CORES * N_SUBCORES

B = 16384
FPT_DIM = 256
DIM = 64
N_ADR = 100000
B_PER_W = B // NW
CHUNK = 128
N_CHUNKS = B_PER_W // CHUNK
TBLK = 2048


def _tc_transpose_body(aet_ref, out_ref):
  y = jnp.transpose(aet_ref[...], (1, 0))
  out_ref[...] = jnp.concatenate(
      [y, jnp.zeros((TBLK, 2 * DIM - DIM), jnp.float32)], axis=1)


def _tc_transpose(aet):
  grid = (pl.cdiv(N_ADR, TBLK),)
  return pl.pallas_call(
      _tc_transpose_body,
      grid=grid,
      in_specs=[pl.BlockSpec((DIM, TBLK), lambda i: (0, i))],
      out_specs=pl.BlockSpec((TBLK, 2 * DIM), lambda i: (i, 0)),
      out_shape=jax.ShapeDtypeStruct((N_ADR, 2 * DIM), jnp.float32),
  )(aet)


def _sc_drug_body(didx_hbm, demb_hbm, drugs_out, didx_v, dbuf0, dbuf1, sem0):
  wid = lax.axis_index("s") * N_CORES + lax.axis_index("c")
  base = wid * B_PER_W

  for k in range(N_CHUNKS):
    pltpu.sync_copy(didx_hbm.at[pl.ds(base + k * CHUNK, CHUNK)], didx_v.at[k])

  bufs = (dbuf0, dbuf1)
  cps = [None] * N_CHUNKS
  cps[0] = pltpu.async_copy(demb_hbm.at[didx_v.at[0]], bufs[0], sem0)
  for k in range(N_CHUNKS):
    if k + 1 < N_CHUNKS:
      cps[k + 1] = pltpu.async_copy(
          demb_hbm.at[didx_v.at[k + 1]], bufs[(k + 1) % 2], sem0)
    cps[k].wait()
    pltpu.sync_copy(bufs[k % 2],
                    drugs_out.at[pl.ds(base + k * CHUNK, CHUNK)])


def _sc_adr_body(didx_hbm, aidx_hbm, atab_hbm, bd_hbm, ba_hbm,
                 adrs_out, bd_out, ba_out,
                 didx_v, aidx_v, abuf, bdbuf, babuf, sem1, sem2):
  wid = lax.axis_index("s") * N_CORES + lax.axis_index("c")
  base = wid * B_PER_W

  for k in range(N_CHUNKS):
    pltpu.sync_copy(didx_hbm.at[pl.ds(base + k * CHUNK, CHUNK)], didx_v.at[k])
    pltpu.sync_copy(aidx_hbm.at[pl.ds(base + k * CHUNK, CHUNK)], aidx_v.at[k])

  cps = []
  for k in range(N_CHUNKS):
    cps.append(pltpu.async_copy(
        atab_hbm.at[aidx_v.at[k]], abuf.at[pl.ds(k * CHUNK, CHUNK)], sem1))
  for k in range(N_CHUNKS):
    cps.append(pltpu.async_copy(
        bd_hbm.at[didx_v.at[k]], bdbuf.at[pl.ds(k * CHUNK, CHUNK)], sem2))
    cps.append(pltpu.async_copy(
        ba_hbm.at[aidx_v.at[k]], babuf.at[pl.ds(k * CHUNK, CHUNK)], sem2))
  for cp in cps:
    cp.wait()
  pltpu.sync_copy(abuf, adrs_out.at[pl.ds(base, B_PER_W)])
  pltpu.sync_copy(bdbuf, bd_out.at[pl.ds(base, B_PER_W)])
  pltpu.sync_copy(babuf, ba_out.at[pl.ds(base, B_PER_W)])


def _sc_gather(drug_idx, adr_idx, drug_embeddings, adr_tab, bias_d, bias_a):
  mesh = plsc.VectorSubcoreMesh(core_axis_name="c", subcore_axis_name="s")

  drug_fn = pl.kernel(
      _sc_drug_body,
      out_type=jax.ShapeDtypeStruct((B, FPT_DIM), jnp.float32),
      mesh=mesh,
      scratch_types=[
          pltpu.VMEM((8, CHUNK), jnp.int32),
          pltpu.VMEM((CHUNK, FPT_DIM), jnp.float32),
          pltpu.VMEM((CHUNK, FPT_DIM), jnp.float32),
          pltpu.SemaphoreType.DMA,
      ],
      compiler_params=pltpu.CompilerParams(use_tc_tiling_on_sc=True))
  drugs_g = drug_fn(drug_idx, drug_embeddings)

  adr_fn = pl.kernel(
      _sc_adr_body,
      out_type=(
          jax.ShapeDtypeStruct((B, 2 * DIM), jnp.float32),
          jax.ShapeDtypeStruct((B,), jnp.float32),
          jax.ShapeDtypeStruct((B,), jnp.float32),
      ),
      mesh=mesh,
      scratch_types=[
          pltpu.VMEM((8, CHUNK), jnp.int32),
          pltpu.VMEM((8, CHUNK), jnp.int32),
          pltpu.VMEM((B_PER_W, 2 * DIM), jnp.float32),
          pltpu.VMEM((B_PER_W,), jnp.float32),
          pltpu.VMEM((B_PER_W,), jnp.float32),
          pltpu.SemaphoreType.DMA,
          pltpu.SemaphoreType.DMA,
      ],
      compiler_params=pltpu.CompilerParams(use_tc_tiling_on_sc=True))
  adrs_g, bd_g, ba_g = adr_fn(drug_idx, adr_idx, adr_tab,
                              bias_d[:, 0], bias_a[:, 0])
  return drugs_g, adrs_g, bd_g, ba_g


def _tc_score_body(drugs_ref, adrs_ref, bd_ref, ba_ref, lw_ref, lb_ref,
                   out_ref):
  proj = lax.dot_general(drugs_ref[...], lw_ref[...],
                         (((1,), (1,)), ((), ())),
                         preferred_element_type=jnp.float32)
  proj = proj + lb_ref[...]
  s = jnp.sum(proj * adrs_ref[...][:, :DIM], axis=1)
  out_ref[...] = s + bd_ref[...] + ba_ref[...]


def _tc_score(drugs_g, adrs_g, bd_g, ba_g, L_w, L_b):
  blk = 2048
  grid = (B // blk,)
  return pl.pallas_call(
      _tc_score_body,
      grid=grid,
      in_specs=[
          pl.BlockSpec((blk, FPT_DIM), lambda i: (i, 0)),
          pl.BlockSpec((blk, 2 * DIM), lambda i: (i, 0)),
          pl.BlockSpec((blk,), lambda i: (i,)),
          pl.BlockSpec((blk,), lambda i: (i,)),
          pl.BlockSpec((DIM, FPT_DIM), lambda i: (0, 0)),
          pl.BlockSpec((1, DIM), lambda i: (0, 0)),
      ],
      out_specs=pl.BlockSpec((blk,), lambda i: (i,)),
      out_shape=jax.ShapeDtypeStruct((B,), jnp.float32),
  )(drugs_g, adrs_g, bd_g, ba_g, L_w, L_b.reshape(1, DIM))


def kernel(drug_idx, adr_idx, drug_embeddings, adr_embeddings, bias_d, bias_a,
           L_w, L_b):
  drug_idx = drug_idx.astype(jnp.int32)
  adr_idx = adr_idx.astype(jnp.int32)
  adr_tab = _tc_transpose(adr_embeddings.T)
  drugs_g, adrs_g, bd_g, ba_g = _sc_gather(
      drug_idx, adr_idx, drug_embeddings, adr_tab, bias_d, bias_a)
  return _tc_score(drugs_g, adrs_g, bd_g, ba_g, L_w, L_b)


# speedup vs baseline: 1.1812x; 1.0758x over previous
import jax
import jax.numpy as jnp
from jax import lax
from jax.experimental import pallas as pl
from jax.experimental.pallas import tpu as pltpu
from jax.experimental.pallas import tpu_sc as plsc

N_CORES = 2
N_SUBCORES = 16
NW = N_CORES * N_SUBCORES

B = 16384
FPT_DIM = 256
DIM = 64
N_ADR = 100000
PAIR_OFF = 51200
B_PER_W = B // NW
CHUNK = 128
N_CHUNKS = B_PER_W // CHUNK
LANES = 16
TBLK = 2048


def _tc_transpose_body(a_ref, b_ref, out_ref):
  out_ref[:, :DIM] = jnp.transpose(a_ref[...], (1, 0))
  out_ref[:, DIM:] = jnp.transpose(b_ref[...], (1, 0))


def _tc_transpose(aet):
  grid = (PAIR_OFF // TBLK,)
  return pl.pallas_call(
      _tc_transpose_body,
      grid=grid,
      in_specs=[
          pl.BlockSpec((DIM, TBLK), lambda i: (0, i)),
          pl.BlockSpec((DIM, TBLK),
                       lambda i: (0, jnp.minimum(i + PAIR_OFF // TBLK,
                                                 N_ADR // TBLK))),
      ],
      out_specs=pl.BlockSpec((TBLK, 2 * DIM), lambda i: (i, 0)),
      out_shape=jax.ShapeDtypeStruct((PAIR_OFF, 2 * DIM), jnp.float32),
  )(aet, aet)


def _sc_drug_body(didx_hbm, demb_hbm, drugs_out, didx_v, dbuf0, dbuf1, sem0):
  wid = lax.axis_index("s") * N_CORES + lax.axis_index("c")
  base = wid * B_PER_W

  for k in range(N_CHUNKS):
    pltpu.sync_copy(didx_hbm.at[pl.ds(base + k * CHUNK, CHUNK)], didx_v.at[k])

  bufs = (dbuf0, dbuf1)
  cps = [None] * N_CHUNKS
  cps[0] = pltpu.async_copy(demb_hbm.at[didx_v.at[0]], bufs[0], sem0)
  for k in range(N_CHUNKS):
    if k + 1 < N_CHUNKS:
      cps[k + 1] = pltpu.async_copy(
          demb_hbm.at[didx_v.at[k + 1]], bufs[(k + 1) % 2], sem0)
    cps[k].wait()
    pltpu.sync_copy(bufs[k % 2],
                    drugs_out.at[pl.ds(base + k * CHUNK, CHUNK)])


def _sc_adr_body(didx_hbm, aidx_hbm, atab_hbm, bd_hbm, ba_hbm,
                 adrs_out, bd_out, ba_out,
                 didx_v, aidx_v, pidx_v, abuf, bdbuf, babuf, sem1, sem2):
  wid = lax.axis_index("s") * N_CORES + lax.axis_index("c")
  base = wid * B_PER_W

  for k in range(N_CHUNKS):
    pltpu.sync_copy(didx_hbm.at[pl.ds(base + k * CHUNK, CHUNK)], didx_v.at[k])
    pltpu.sync_copy(aidx_hbm.at[pl.ds(base + k * CHUNK, CHUNK)], aidx_v.at[k])

  for k in range(N_CHUNKS):
    for j in range(CHUNK // LANES):
      sl = pl.ds(j * LANES, LANES)
      a = aidx_v[k, sl]
      pidx_v[k, sl] = jnp.where(a >= PAIR_OFF, a - PAIR_OFF, a)

  cps = []
  for k in range(N_CHUNKS):
    cps.append(pltpu.async_copy(
        atab_hbm.at[pidx_v.at[k]], abuf.at[pl.ds(k * CHUNK, CHUNK)], sem1))
  for k in range(N_CHUNKS):
    cps.append(pltpu.async_copy(
        bd_hbm.at[didx_v.at[k]], bdbuf.at[pl.ds(k * CHUNK, CHUNK)], sem2))
    cps.append(pltpu.async_copy(
        ba_hbm.at[aidx_v.at[k]], babuf.at[pl.ds(k * CHUNK, CHUNK)], sem2))
  for cp in cps:
    cp.wait()
  pltpu.sync_copy(abuf, adrs_out.at[pl.ds(base, B_PER_W)])
  pltpu.sync_copy(bdbuf, bd_out.at[pl.ds(base, B_PER_W)])
  pltpu.sync_copy(babuf, ba_out.at[pl.ds(base, B_PER_W)])


_MESH = plsc.VectorSubcoreMesh(core_axis_name="c", subcore_axis_name="s")

_DRUG_FN = pl.kernel(
    _sc_drug_body,
    out_type=jax.ShapeDtypeStruct((B, FPT_DIM), jnp.float32),
    mesh=_MESH,
    scratch_types=[
        pltpu.VMEM((8, CHUNK), jnp.int32),
        pltpu.VMEM((CHUNK, FPT_DIM), jnp.float32),
        pltpu.VMEM((CHUNK, FPT_DIM), jnp.float32),
        pltpu.SemaphoreType.DMA,
    ],
    compiler_params=pltpu.CompilerParams(use_tc_tiling_on_sc=True))

_ADR_FN = pl.kernel(
    _sc_adr_body,
    out_type=(
        jax.ShapeDtypeStruct((B, 2 * DIM), jnp.float32),
        jax.ShapeDtypeStruct((B,), jnp.float32),
        jax.ShapeDtypeStruct((B,), jnp.float32),
    ),
    mesh=_MESH,
    scratch_types=[
        pltpu.VMEM((8, CHUNK), jnp.int32),
        pltpu.VMEM((8, CHUNK), jnp.int32),
        pltpu.VMEM((8, CHUNK), jnp.int32),
        pltpu.VMEM((B_PER_W, 2 * DIM), jnp.float32),
        pltpu.VMEM((B_PER_W,), jnp.float32),
        pltpu.VMEM((B_PER_W,), jnp.float32),
        pltpu.SemaphoreType.DMA,
        pltpu.SemaphoreType.DMA,
    ],
    compiler_params=pltpu.CompilerParams(use_tc_tiling_on_sc=True))


def _tc_score_body(drugs_ref, pairs_ref, aidx_ref, bd_ref, ba_ref, lw_ref,
                   lb_ref, out_ref):
  proj = lax.dot_general(drugs_ref[...], lw_ref[...],
                         (((1,), (1,)), ((), ())),
                         preferred_element_type=jnp.float32)
  proj = proj + lb_ref[...]
  pair = pairs_ref[...]
  s_lo = jnp.sum(proj * pair[:, :DIM], axis=1)
  s_hi = jnp.sum(proj * pair[:, DIM:], axis=1)
  s = jnp.where(aidx_ref[...] >= PAIR_OFF, s_hi, s_lo)
  out_ref[...] = s + bd_ref[...] + ba_ref[...]


def _tc_score(drugs_g, pairs_g, adr_idx, bd_g, ba_g, L_w, L_b):
  blk = 2048
  grid = (B // blk,)
  return pl.pallas_call(
      _tc_score_body,
      grid=grid,
      in_specs=[
          pl.BlockSpec((blk, FPT_DIM), lambda i: (i, 0)),
          pl.BlockSpec((blk, 2 * DIM), lambda i: (i, 0)),
          pl.BlockSpec((blk,), lambda i: (i,)),
          pl.BlockSpec((blk,), lambda i: (i,)),
          pl.BlockSpec((blk,), lambda i: (i,)),
          pl.BlockSpec((DIM, FPT_DIM), lambda i: (0, 0)),
          pl.BlockSpec((1, DIM), lambda i: (0, 0)),
      ],
      out_specs=pl.BlockSpec((blk,), lambda i: (i,)),
      out_shape=jax.ShapeDtypeStruct((B,), jnp.float32),
  )(drugs_g, pairs_g, adr_idx, bd_g, ba_g, L_w, L_b.reshape(1, DIM))


def kernel(drug_idx, adr_idx, drug_embeddings, adr_embeddings, bias_d, bias_a,
           L_w, L_b):
  drug_idx = drug_idx.astype(jnp.int32)
  adr_idx = adr_idx.astype(jnp.int32)
  drugs_g = _DRUG_FN(drug_idx, drug_embeddings)
  bd_flat = bias_d[:, 0]
  ba_flat = bias_a[:, 0]
  adr_tab = _tc_transpose(adr_embeddings.T)
  pairs_g, bd_g, ba_g = _ADR_FN(drug_idx, adr_idx, adr_tab, bd_flat, ba_flat)
  return _tc_score(drugs_g, pairs_g, adr_idx, bd_g, ba_g, L_w, L_b)


# speedup vs baseline: 1.3348x; 1.1300x over previous
import jax
import jax.numpy as jnp
from jax import lax
from jax.experimental import pallas as pl
from jax.experimental.pallas import tpu as pltpu
from jax.experimental.pallas import tpu_sc as plsc

N_CORES = 2
N_SUBCORES = 16
NW = N_CORES * N_SUBCORES

B = 16384
BH = B // 2
FPT_DIM = 256
DIM = 64
N_ADR = 100000
PAIR_OFF = 51200
B_PER_W = B // NW
BH_PER_W = BH // NW
CHUNK = 128
N_CHUNKS = B_PER_W // CHUNK
NH_CHUNKS = BH_PER_W // CHUNK
LANES = 16
TBLK = 2048


def _tc_transpose_body(a_ref, b_ref, out_ref):
  out_ref[:, :DIM] = jnp.transpose(a_ref[...], (1, 0))
  out_ref[:, DIM:] = jnp.transpose(b_ref[...], (1, 0))


def _tc_transpose(aet):
  grid = (PAIR_OFF // TBLK,)
  return pl.pallas_call(
      _tc_transpose_body,
      grid=grid,
      in_specs=[
          pl.BlockSpec((DIM, TBLK), lambda i: (0, i)),
          pl.BlockSpec((DIM, TBLK),
                       lambda i: (0, jnp.minimum(i + PAIR_OFF // TBLK,
                                                 N_ADR // TBLK))),
      ],
      out_specs=pl.BlockSpec((TBLK, 2 * DIM), lambda i: (i, 0)),
      out_shape=jax.ShapeDtypeStruct((PAIR_OFF, 2 * DIM), jnp.float32),
  )(aet, aet)


def _sc_drug_body(didx_hbm, demb_hbm, drugs_out, didx_v, dbuf0, dbuf1, sem0):
  wid = lax.axis_index("s") * N_CORES + lax.axis_index("c")
  base = wid * B_PER_W

  for k in range(N_CHUNKS):
    pltpu.sync_copy(didx_hbm.at[pl.ds(base + k * CHUNK, CHUNK)], didx_v.at[k])

  bufs = (dbuf0, dbuf1)
  cps = [None] * N_CHUNKS
  cps[0] = pltpu.async_copy(demb_hbm.at[didx_v.at[0]], bufs[0], sem0)
  for k in range(N_CHUNKS):
    if k + 1 < N_CHUNKS:
      cps[k + 1] = pltpu.async_copy(
          demb_hbm.at[didx_v.at[k + 1]], bufs[(k + 1) % 2], sem0)
    cps[k].wait()
    pltpu.sync_copy(bufs[k % 2],
                    drugs_out.at[pl.ds(base + k * CHUNK, CHUNK)])


def _make_adr_body(half):
  hbase = half * BH

  def _sc_adr_body(didx_hbm, aidx_hbm, atab_hbm, bd_hbm, ba_hbm,
                   adrs_out, bd_out, ba_out,
                   didx_v, aidx_v, pidx_v, abuf, bdbuf, babuf, sem1, sem2):
    wid = lax.axis_index("s") * N_CORES + lax.axis_index("c")
    base = wid * BH_PER_W
    src = hbase + base

    for k in range(NH_CHUNKS):
      pltpu.sync_copy(didx_hbm.at[pl.ds(src + k * CHUNK, CHUNK)],
                      didx_v.at[k])
      pltpu.sync_copy(aidx_hbm.at[pl.ds(src + k * CHUNK, CHUNK)],
                      aidx_v.at[k])

    for k in range(NH_CHUNKS):
      for j in range(CHUNK // LANES):
        sl = pl.ds(j * LANES, LANES)
        a = aidx_v[k, sl]
        pidx_v[k, sl] = jnp.where(a >= PAIR_OFF, a - PAIR_OFF, a)

    cps = []
    for k in range(NH_CHUNKS):
      cps.append(pltpu.async_copy(
          atab_hbm.at[pidx_v.at[k]], abuf.at[pl.ds(k * CHUNK, CHUNK)], sem1))
    for k in range(NH_CHUNKS):
      cps.append(pltpu.async_copy(
          bd_hbm.at[didx_v.at[k]], bdbuf.at[pl.ds(k * CHUNK, CHUNK)], sem2))
      cps.append(pltpu.async_copy(
          ba_hbm.at[aidx_v.at[k]], babuf.at[pl.ds(k * CHUNK, CHUNK)], sem2))
    for cp in cps:
      cp.wait()
    pltpu.sync_copy(abuf, adrs_out.at[pl.ds(base, BH_PER_W)])
    pltpu.sync_copy(bdbuf, bd_out.at[pl.ds(base, BH_PER_W)])
    pltpu.sync_copy(babuf, ba_out.at[pl.ds(base, BH_PER_W)])

  return _sc_adr_body


_MESH = plsc.VectorSubcoreMesh(core_axis_name="c", subcore_axis_name="s")

_DRUG_FN = pl.kernel(
    _sc_drug_body,
    out_type=jax.ShapeDtypeStruct((B, FPT_DIM), jnp.float32),
    mesh=_MESH,
    scratch_types=[
        pltpu.VMEM((8, CHUNK), jnp.int32),
        pltpu.VMEM((CHUNK, FPT_DIM), jnp.float32),
        pltpu.VMEM((CHUNK, FPT_DIM), jnp.float32),
        pltpu.SemaphoreType.DMA,
    ],
    compiler_params=pltpu.CompilerParams(use_tc_tiling_on_sc=True))


def _make_adr_fn(half):
  return pl.kernel(
      _make_adr_body(half),
      out_type=(
          jax.ShapeDtypeStruct((BH, 2 * DIM), jnp.float32),
          jax.ShapeDtypeStruct((BH,), jnp.float32),
          jax.ShapeDtypeStruct((BH,), jnp.float32),
      ),
      mesh=_MESH,
      scratch_types=[
          pltpu.VMEM((8, CHUNK), jnp.int32),
          pltpu.VMEM((8, CHUNK), jnp.int32),
          pltpu.VMEM((8, CHUNK), jnp.int32),
          pltpu.VMEM((BH_PER_W, 2 * DIM), jnp.float32),
          pltpu.VMEM((BH_PER_W,), jnp.float32),
          pltpu.VMEM((BH_PER_W,), jnp.float32),
          pltpu.SemaphoreType.DMA,
          pltpu.SemaphoreType.DMA,
      ],
      compiler_params=pltpu.CompilerParams(use_tc_tiling_on_sc=True))


_ADR_FN = (_make_adr_fn(0), _make_adr_fn(1))


def _tc_score_body(drugs_ref, pairs_ref, aidx_ref, bd_ref, ba_ref, lw_ref,
                   lb_ref, out_ref):
  proj = lax.dot_general(drugs_ref[...], lw_ref[...],
                         (((1,), (1,)), ((), ())),
                         preferred_element_type=jnp.float32)
  proj = proj + lb_ref[...]
  pair = pairs_ref[...]
  s_lo = jnp.sum(proj * pair[:, :DIM], axis=1)
  s_hi = jnp.sum(proj * pair[:, DIM:], axis=1)
  s = jnp.where(aidx_ref[...] >= PAIR_OFF, s_hi, s_lo)
  out_ref[...] = s + bd_ref[...] + ba_ref[...]


def _tc_score(half, drugs_g, pairs_h, adr_idx, bd_h, ba_h, L_w, L_b):
  blk = 2048
  grid = (BH // blk,)
  hblk = half * (BH // blk)
  return pl.pallas_call(
      _tc_score_body,
      grid=grid,
      in_specs=[
          pl.BlockSpec((blk, FPT_DIM), lambda i: (i + hblk, 0)),
          pl.BlockSpec((blk, 2 * DIM), lambda i: (i, 0)),
          pl.BlockSpec((blk,), lambda i: (i + hblk,)),
          pl.BlockSpec((blk,), lambda i: (i,)),
          pl.BlockSpec((blk,), lambda i: (i,)),
          pl.BlockSpec((DIM, FPT_DIM), lambda i: (0, 0)),
          pl.BlockSpec((1, DIM), lambda i: (0, 0)),
      ],
      out_specs=pl.BlockSpec((blk,), lambda i: (i,)),
      out_shape=jax.ShapeDtypeStruct((BH,), jnp.float32),
  )(drugs_g, pairs_h, adr_idx, bd_h, ba_h, L_w, L_b.reshape(1, DIM))


def kernel(drug_idx, adr_idx, drug_embeddings, adr_embeddings, bias_d, bias_a,
           L_w, L_b):
  drug_idx = drug_idx.astype(jnp.int32)
  adr_idx = adr_idx.astype(jnp.int32)
  drugs_g = _DRUG_FN(drug_idx, drug_embeddings)
  bd_flat = bias_d[:, 0]
  ba_flat = bias_a[:, 0]
  adr_tab = _tc_transpose(adr_embeddings.T)
  pairs_0, bd_0, ba_0 = _ADR_FN[0](drug_idx, adr_idx, adr_tab, bd_flat,
                                   ba_flat)
  pairs_1, bd_1, ba_1 = _ADR_FN[1](drug_idx, adr_idx, adr_tab, bd_flat,
                                   ba_flat)
  s0 = _tc_score(0, drugs_g, pairs_0, adr_idx, bd_0, ba_0, L_w, L_b)
  s1 = _tc_score(1, drugs_g, pairs_1, adr_idx, bd_1, ba_1, L_w, L_b)
  return jnp.concatenate([s0, s1])


# speedup vs baseline: 1.4271x; 1.0692x over previous
import jax
import jax.numpy as jnp
from jax import lax
from jax.experimental import pallas as pl
from jax.experimental.pallas import tpu as pltpu
from jax.experimental.pallas import tpu_sc as plsc

N_CORES = 2
N_SUBCORES = 16
NW = N_CORES * N_SUBCORES

B = 16384
BH = B // 2
FPT_DIM = 256
DIM = 64
N_ADR = 100000
PAIR_OFF = 51200
B_PER_W = B // NW
BH_PER_W = BH // NW
CHUNK = 128
N_CHUNKS = B_PER_W // CHUNK
NH_CHUNKS = BH_PER_W // CHUNK
LANES = 16
TBLK = 5120


def _tc_transpose_body(a_ref, b_ref, out_ref):
  out_ref[:, :DIM] = jnp.transpose(a_ref[...], (1, 0))
  out_ref[:, DIM:] = jnp.transpose(b_ref[...], (1, 0))


def _tc_transpose(aet):
  grid = (PAIR_OFF // TBLK,)
  return pl.pallas_call(
      _tc_transpose_body,
      grid=grid,
      in_specs=[
          pl.BlockSpec((DIM, TBLK), lambda i: (0, i)),
          pl.BlockSpec((DIM, TBLK),
                       lambda i: (0, jnp.minimum(i + PAIR_OFF // TBLK,
                                                 N_ADR // TBLK))),
      ],
      out_specs=pl.BlockSpec((TBLK, 2 * DIM), lambda i: (i, 0)),
      out_shape=jax.ShapeDtypeStruct((PAIR_OFF, 2 * DIM), jnp.float32),
  )(aet, aet)


def _sc_drug_body(didx_hbm, demb_hbm, drugs_out, didx_v, dbuf0, dbuf1, sem0):
  wid = lax.axis_index("s") * N_CORES + lax.axis_index("c")
  base = wid * B_PER_W

  for k in range(N_CHUNKS):
    pltpu.sync_copy(didx_hbm.at[pl.ds(base + k * CHUNK, CHUNK)], didx_v.at[k])

  bufs = (dbuf0, dbuf1)
  cps = [None] * N_CHUNKS
  cps[0] = pltpu.async_copy(demb_hbm.at[didx_v.at[0]], bufs[0], sem0)
  for k in range(N_CHUNKS):
    if k + 1 < N_CHUNKS:
      cps[k + 1] = pltpu.async_copy(
          demb_hbm.at[didx_v.at[k + 1]], bufs[(k + 1) % 2], sem0)
    cps[k].wait()
    pltpu.sync_copy(bufs[k % 2],
                    drugs_out.at[pl.ds(base + k * CHUNK, CHUNK)])


def _make_adr_body(half):
  hbase = half * BH

  def _sc_adr_body(didx_hbm, aidx_hbm, atab_hbm, bd_hbm, ba_hbm,
                   adrs_out, bd_out, ba_out,
                   didx_v, aidx_v, pidx_v, abuf, bdbuf, babuf, sem1, sem2):
    wid = lax.axis_index("s") * N_CORES + lax.axis_index("c")
    base = wid * BH_PER_W
    src = hbase + base

    for k in range(NH_CHUNKS):
      pltpu.sync_copy(didx_hbm.at[pl.ds(src + k * CHUNK, CHUNK)],
                      didx_v.at[k])
      pltpu.sync_copy(aidx_hbm.at[pl.ds(src + k * CHUNK, CHUNK)],
                      aidx_v.at[k])

    for k in range(NH_CHUNKS):
      for j in range(CHUNK // LANES):
        sl = pl.ds(j * LANES, LANES)
        a = aidx_v[k, sl]
        pidx_v[k, sl] = jnp.where(a >= PAIR_OFF, a - PAIR_OFF, a)

    cps = []
    for k in range(NH_CHUNKS):
      cps.append(pltpu.async_copy(
          atab_hbm.at[pidx_v.at[k]], abuf.at[pl.ds(k * CHUNK, CHUNK)], sem1))
    for k in range(NH_CHUNKS):
      cps.append(pltpu.async_copy(
          bd_hbm.at[didx_v.at[k]], bdbuf.at[pl.ds(k * CHUNK, CHUNK)], sem2))
      cps.append(pltpu.async_copy(
          ba_hbm.at[aidx_v.at[k]], babuf.at[pl.ds(k * CHUNK, CHUNK)], sem2))
    for cp in cps:
      cp.wait()
    pltpu.sync_copy(abuf, adrs_out.at[pl.ds(base, BH_PER_W)])
    pltpu.sync_copy(bdbuf, bd_out.at[pl.ds(base, BH_PER_W)])
    pltpu.sync_copy(babuf, ba_out.at[pl.ds(base, BH_PER_W)])

  return _sc_adr_body


_MESH = plsc.VectorSubcoreMesh(core_axis_name="c", subcore_axis_name="s")

_DRUG_FN = pl.kernel(
    _sc_drug_body,
    out_type=jax.ShapeDtypeStruct((B, FPT_DIM), jnp.float32),
    mesh=_MESH,
    scratch_types=[
        pltpu.VMEM((8, CHUNK), jnp.int32),
        pltpu.VMEM((CHUNK, FPT_DIM), jnp.float32),
        pltpu.VMEM((CHUNK, FPT_DIM), jnp.float32),
        pltpu.SemaphoreType.DMA,
    ],
    compiler_params=pltpu.CompilerParams(use_tc_tiling_on_sc=True))


def _make_adr_fn(half):
  return pl.kernel(
      _make_adr_body(half),
      out_type=(
          jax.ShapeDtypeStruct((BH, 2 * DIM), jnp.float32),
          jax.ShapeDtypeStruct((BH,), jnp.float32),
          jax.ShapeDtypeStruct((BH,), jnp.float32),
      ),
      mesh=_MESH,
      scratch_types=[
          pltpu.VMEM((8, CHUNK), jnp.int32),
          pltpu.VMEM((8, CHUNK), jnp.int32),
          pltpu.VMEM((8, CHUNK), jnp.int32),
          pltpu.VMEM((BH_PER_W, 2 * DIM), jnp.float32),
          pltpu.VMEM((BH_PER_W,), jnp.float32),
          pltpu.VMEM((BH_PER_W,), jnp.float32),
          pltpu.SemaphoreType.DMA,
          pltpu.SemaphoreType.DMA,
      ],
      compiler_params=pltpu.CompilerParams(use_tc_tiling_on_sc=True))


_ADR_FN = (_make_adr_fn(0), _make_adr_fn(1))


def _tc_score_body(drugs_ref, pairs_ref, aidx_ref, bd_ref, ba_ref, lw_ref,
                   lb_ref, out_ref):
  proj = lax.dot_general(drugs_ref[...], lw_ref[...],
                         (((1,), (1,)), ((), ())),
                         preferred_element_type=jnp.float32)
  proj = proj + lb_ref[...]
  pair = pairs_ref[...]
  s_lo = jnp.sum(proj * pair[:, :DIM], axis=1)
  s_hi = jnp.sum(proj * pair[:, DIM:], axis=1)
  s = jnp.where(aidx_ref[...] >= PAIR_OFF, s_hi, s_lo)
  out_ref[...] = s + bd_ref[...] + ba_ref[...]


def _tc_score(half, drugs_g, pairs_h, adr_idx, bd_h, ba_h, L_w, L_b):
  blk = 4096
  grid = (BH // blk,)
  hblk = half * (BH // blk)
  return pl.pallas_call(
      _tc_score_body,
      grid=grid,
      in_specs=[
          pl.BlockSpec((blk, FPT_DIM), lambda i: (i + hblk, 0)),
          pl.BlockSpec((blk, 2 * DIM), lambda i: (i, 0)),
          pl.BlockSpec((blk,), lambda i: (i + hblk,)),
          pl.BlockSpec((blk,), lambda i: (i,)),
          pl.BlockSpec((blk,), lambda i: (i,)),
          pl.BlockSpec((DIM, FPT_DIM), lambda i: (0, 0)),
          pl.BlockSpec((1, DIM), lambda i: (0, 0)),
      ],
      out_specs=pl.BlockSpec((blk,), lambda i: (i,)),
      out_shape=jax.ShapeDtypeStruct((BH,), jnp.float32),
  )(drugs_g, pairs_h, adr_idx, bd_h, ba_h, L_w, L_b.reshape(1, DIM))


def kernel(drug_idx, adr_idx, drug_embeddings, adr_embeddings, bias_d, bias_a,
           L_w, L_b):
  drug_idx = drug_idx.astype(jnp.int32)
  adr_idx = adr_idx.astype(jnp.int32)
  drugs_g = _DRUG_FN(drug_idx, drug_embeddings)
  bd_flat = bias_d[:, 0]
  ba_flat = bias_a[:, 0]
  adr_tab = _tc_transpose(adr_embeddings.T)
  pairs_0, bd_0, ba_0 = _ADR_FN[0](drug_idx, adr_idx, adr_tab, bd_flat,
                                   ba_flat)
  pairs_1, bd_1, ba_1 = _ADR_FN[1](drug_idx, adr_idx, adr_tab, bd_flat,
                                   ba_flat)
  s0 = _tc_score(0, drugs_g, pairs_0, adr_idx, bd_0, ba_0, L_w, L_b)
  s1 = _tc_score(1, drugs_g, pairs_1, adr_idx, bd_1, ba_1, L_w, L_b)
  return jnp.concatenate([s0, s1])


# speedup vs baseline: 1.4499x; 1.0160x over previous
import jax
import jax.numpy as jnp
from jax import lax
from jax.experimental import pallas as pl
from jax.experimental.pallas import tpu as pltpu
from jax.experimental.pallas import tpu_sc as plsc

N_CORES = 2
N_SUBCORES = 16
NW = N_CORES * N_SUBCORES

B = 16384
BH = B // 2
FPT_DIM = 256
DIM = 64
N_ADR = 100000
PAIR_OFF = 51200
B_PER_W = B // NW
BH_PER_W = BH // NW
CHUNK = 128
N_CHUNKS = B_PER_W // CHUNK
NH_CHUNKS = BH_PER_W // CHUNK
LANES = 16
TBLK = 10240


def _tc_transpose_body(a_ref, b_ref, out_ref):
  out_ref[:, :DIM] = jnp.transpose(a_ref[...], (1, 0))
  out_ref[:, DIM:] = jnp.transpose(b_ref[...], (1, 0))


def _tc_transpose(aet):
  grid = (PAIR_OFF // TBLK,)
  return pl.pallas_call(
      _tc_transpose_body,
      grid=grid,
      in_specs=[
          pl.BlockSpec((DIM, TBLK), lambda i: (0, i)),
          pl.BlockSpec((DIM, TBLK),
                       lambda i: (0, jnp.minimum(i + PAIR_OFF // TBLK,
                                                 N_ADR // TBLK))),
      ],
      out_specs=pl.BlockSpec((TBLK, 2 * DIM), lambda i: (i, 0)),
      out_shape=jax.ShapeDtypeStruct((PAIR_OFF, 2 * DIM), jnp.float32),
  )(aet, aet)


def _sc_drug_body(didx_hbm, demb_hbm, drugs_out, didx_v, dbuf0, dbuf1, sem0):
  wid = lax.axis_index("s") * N_CORES + lax.axis_index("c")
  base = wid * B_PER_W

  for k in range(N_CHUNKS):
    pltpu.sync_copy(didx_hbm.at[pl.ds(base + k * CHUNK, CHUNK)], didx_v.at[k])

  bufs = (dbuf0, dbuf1)
  cps = [None] * N_CHUNKS
  cps[0] = pltpu.async_copy(demb_hbm.at[didx_v.at[0]], bufs[0], sem0)
  for k in range(N_CHUNKS):
    if k + 1 < N_CHUNKS:
      cps[k + 1] = pltpu.async_copy(
          demb_hbm.at[didx_v.at[k + 1]], bufs[(k + 1) % 2], sem0)
    cps[k].wait()
    pltpu.sync_copy(bufs[k % 2],
                    drugs_out.at[pl.ds(base + k * CHUNK, CHUNK)])


def _make_adr_body(half):
  hbase = half * BH

  def _sc_adr_body(didx_hbm, aidx_hbm, atab_hbm, bd_hbm, ba_hbm,
                   adrs_out, bd_out, ba_out,
                   didx_v, aidx_v, pidx_v, abuf, bdbuf, babuf, sem1, sem2):
    wid = lax.axis_index("s") * N_CORES + lax.axis_index("c")
    base = wid * BH_PER_W
    src = hbase + base

    for k in range(NH_CHUNKS):
      pltpu.sync_copy(didx_hbm.at[pl.ds(src + k * CHUNK, CHUNK)],
                      didx_v.at[k])
      pltpu.sync_copy(aidx_hbm.at[pl.ds(src + k * CHUNK, CHUNK)],
                      aidx_v.at[k])

    for k in range(NH_CHUNKS):
      for j in range(CHUNK // LANES):
        sl = pl.ds(j * LANES, LANES)
        a = aidx_v[k, sl]
        pidx_v[k, sl] = jnp.where(a >= PAIR_OFF, a - PAIR_OFF, a)

    cps = []
    for k in range(NH_CHUNKS):
      cps.append(pltpu.async_copy(
          atab_hbm.at[pidx_v.at[k]], abuf.at[pl.ds(k * CHUNK, CHUNK)], sem1))
    for k in range(NH_CHUNKS):
      cps.append(pltpu.async_copy(
          bd_hbm.at[didx_v.at[k]], bdbuf.at[pl.ds(k * CHUNK, CHUNK)], sem2))
      cps.append(pltpu.async_copy(
          ba_hbm.at[aidx_v.at[k]], babuf.at[pl.ds(k * CHUNK, CHUNK)], sem2))
    for cp in cps:
      cp.wait()
    pltpu.sync_copy(abuf, adrs_out.at[pl.ds(base, BH_PER_W)])
    pltpu.sync_copy(bdbuf, bd_out.at[pl.ds(base, BH_PER_W)])
    pltpu.sync_copy(babuf, ba_out.at[pl.ds(base, BH_PER_W)])

  return _sc_adr_body


_MESH = plsc.VectorSubcoreMesh(core_axis_name="c", subcore_axis_name="s")

_DRUG_FN = pl.kernel(
    _sc_drug_body,
    out_type=jax.ShapeDtypeStruct((B, FPT_DIM), jnp.float32),
    mesh=_MESH,
    scratch_types=[
        pltpu.VMEM((8, CHUNK), jnp.int32),
        pltpu.VMEM((CHUNK, FPT_DIM), jnp.float32),
        pltpu.VMEM((CHUNK, FPT_DIM), jnp.float32),
        pltpu.SemaphoreType.DMA,
    ],
    compiler_params=pltpu.CompilerParams(use_tc_tiling_on_sc=True))


def _make_adr_fn(half):
  return pl.kernel(
      _make_adr_body(half),
      out_type=(
          jax.ShapeDtypeStruct((BH, 2 * DIM), jnp.float32),
          jax.ShapeDtypeStruct((BH,), jnp.float32),
          jax.ShapeDtypeStruct((BH,), jnp.float32),
      ),
      mesh=_MESH,
      scratch_types=[
          pltpu.VMEM((8, CHUNK), jnp.int32),
          pltpu.VMEM((8, CHUNK), jnp.int32),
          pltpu.VMEM((8, CHUNK), jnp.int32),
          pltpu.VMEM((BH_PER_W, 2 * DIM), jnp.float32),
          pltpu.VMEM((BH_PER_W,), jnp.float32),
          pltpu.VMEM((BH_PER_W,), jnp.float32),
          pltpu.SemaphoreType.DMA,
          pltpu.SemaphoreType.DMA,
      ],
      compiler_params=pltpu.CompilerParams(use_tc_tiling_on_sc=True))


_ADR_FN = (_make_adr_fn(0), _make_adr_fn(1))


def _tc_score_body(drugs_ref, pairs_ref, aidx_ref, bd_ref, ba_ref, lw_ref,
                   lb_ref, out_ref):
  proj = lax.dot_general(drugs_ref[...], lw_ref[...],
                         (((1,), (1,)), ((), ())),
                         preferred_element_type=jnp.float32)
  proj = proj + lb_ref[...]
  pair = pairs_ref[...]
  s_lo = jnp.sum(proj * pair[:, :DIM], axis=1)
  s_hi = jnp.sum(proj * pair[:, DIM:], axis=1)
  s = jnp.where(aidx_ref[...] >= PAIR_OFF, s_hi, s_lo)
  out_ref[...] = s + bd_ref[...] + ba_ref[...]


def _tc_score(half, drugs_g, pairs_h, adr_idx, bd_h, ba_h, L_w, L_b):
  blk = 4096
  grid = (BH // blk,)
  hblk = half * (BH // blk)
  return pl.pallas_call(
      _tc_score_body,
      grid=grid,
      in_specs=[
          pl.BlockSpec((blk, FPT_DIM), lambda i: (i + hblk, 0)),
          pl.BlockSpec((blk, 2 * DIM), lambda i: (i, 0)),
          pl.BlockSpec((blk,), lambda i: (i + hblk,)),
          pl.BlockSpec((blk,), lambda i: (i,)),
          pl.BlockSpec((blk,), lambda i: (i,)),
          pl.BlockSpec((DIM, FPT_DIM), lambda i: (0, 0)),
          pl.BlockSpec((1, DIM), lambda i: (0, 0)),
      ],
      out_specs=pl.BlockSpec((blk,), lambda i: (i,)),
      out_shape=jax.ShapeDtypeStruct((BH,), jnp.float32),
  )(drugs_g, pairs_h, adr_idx, bd_h, ba_h, L_w, L_b.reshape(1, DIM))


def kernel(drug_idx, adr_idx, drug_embeddings, adr_embeddings, bias_d, bias_a,
           L_w, L_b):
  drug_idx = drug_idx.astype(jnp.int32)
  adr_idx = adr_idx.astype(jnp.int32)
  drugs_g = _DRUG_FN(drug_idx, drug_embeddings)
  bd_flat = bias_d[:, 0]
  ba_flat = bias_a[:, 0]
  adr_tab = _tc_transpose(adr_embeddings.T)
  pairs_0, bd_0, ba_0 = _ADR_FN[0](drug_idx, adr_idx, adr_tab, bd_flat,
                                   ba_flat)
  pairs_1, bd_1, ba_1 = _ADR_FN[1](drug_idx, adr_idx, adr_tab, bd_flat,
                                   ba_flat)
  s0 = _tc_score(0, drugs_g, pairs_0, adr_idx, bd_0, ba_0, L_w, L_b)
  s1 = _tc_score(1, drugs_g, pairs_1, adr_idx, bd_1, ba_1, L_w, L_b)
  return jnp.concatenate([s0, s1])
